# Initial kernel scaffold; baseline (speedup 1.0000x reference)
import jax
import jax.numpy as jnp
from jax.experimental import pallas as pl


def kernel(z, edge_index, edge_weight, edge_vec_norm, edge_attr, labels, labels_mask, emb_w, emb2_w, emb2_b, label_emb_w, label_emb_b, label_emb2_w, label_emb2_b, mask_token, dp1_w, dp1_b, dp2_w, dp2_b, dp3_w, dp3_b, lt0_w, lt1_w, lt2_w, ls0_w, ls0_b, ls1_w, ls1_b, ln_g, ln_b):
    raise NotImplementedError("write your pallas kernel here")


# speedup vs baseline: 16.9275x; 16.9275x over previous
import functools

import jax
import jax.numpy as jnp
import numpy as np
from jax import lax
from jax.experimental import pallas as pl
from jax.experimental.pallas import tpu as pltpu
from jax.experimental.pallas import tpu_sc as plsc

N = 10000
E = 160000
H = 128
R = 16

NC = 2
NS = 16
CH = 8
HC = 16
NCOMP = 9

NW = NC * NS
ET = E // NW
KB = 40
NB_BLK = ET // KB
KS = KB
NPAD = 10240
NPT = NPAD // NS

_HIGH = jax.lax.Precision.HIGHEST


def _dot(a, b):
    return jax.lax.dot_general(a, b, (((1,), (0,)), ((), ())),
                               precision=_HIGH,
                               preferred_element_type=jnp.float32)


def _node_tables_kernel(z_ref, lab_ref, msk_ref, emb_ref, lwT_ref, leb_ref,
                        mtok_ref, e2sT_ref, e2dT_ref, l2sT_ref, l2dT_ref,
                        bq_ref, psrc_ref, pdst_ref):
    zb = z_ref[...]
    oh = (zb == lax.broadcasted_iota(jnp.int32, (zb.shape[0], 128), 1))
    zemb = _dot(oh.astype(jnp.float32), emb_ref[...])
    lab = lab_ref[...]
    m = msk_ref[...]
    L = lab * lwT_ref[...] + leb_ref[...]
    L = L * (1.0 - m) + mtok_ref[...] * m
    psrc_ref[...] = _dot(zemb, e2sT_ref[...]) + _dot(L, l2sT_ref[...]) + bq_ref[...]
    pdst_ref[...] = _dot(zemb, e2dT_ref[...]) + _dot(L, l2dT_ref[...])


def _node_tables(z2, lab2, msk2, emb_w, lwT, leb, mtok, e2sT, e2dT, l2sT,
                 l2dT, bq):
    nb = 1000
    grid = (N // nb,)
    full = lambda shape: pl.BlockSpec(shape, lambda i: (0,) * len(shape))
    return pl.pallas_call(
        _node_tables_kernel,
        grid=grid,
        in_specs=[
            pl.BlockSpec((nb, 1), lambda i: (i, 0)),
            pl.BlockSpec((nb, 1), lambda i: (i, 0)),
            pl.BlockSpec((nb, 1), lambda i: (i, 0)),
            full((128, H)), full((1, H)), full((1, H)), full((1, H)),
            full((H, H)), full((H, H)), full((H, H)), full((H, H)),
            full((1, H)),
        ],
        out_specs=[pl.BlockSpec((nb, H), lambda i: (i, 0)),
                   pl.BlockSpec((nb, H), lambda i: (i, 0))],
        out_shape=[jax.ShapeDtypeStruct((N, H), jnp.float32),
                   jax.ShapeDtypeStruct((N, H), jnp.float32)],
    )(z2, lab2, msk2, emb_w, lwT, leb, mtok, e2sT, e2dT, l2sT, l2dT, bq)


def _edge_w_kernel(attr_ref, ew_ref, v0_ref, v1_ref, v2_ref, dpwT_ref,
                   dpb_ref, w_ref):
    attr = attr_ref[...]
    D = _dot(attr, dpwT_ref[...]) + dpb_ref[...]
    d1 = D[:, :H]
    d2 = D[:, H:2 * H]
    d3 = D[:, 2 * H:]
    ew = ew_ref[...]
    c = 0.5 * (jnp.cos(ew * jnp.float32(np.pi / 5.0)) + 1.0)
    c = jnp.where(ew < 5.0, c, 0.0)
    v0 = v0_ref[...]
    v1 = v1_ref[...]
    v2 = v2_ref[...]
    w0 = d1 * c
    cd2 = d2 * c
    cd3 = d3 * c
    s3 = (v0 * v0 + v1 * v1 + v2 * v2) * jnp.float32(1.0 / 3.0)
    w_ref[...] = jnp.stack([
        w0,
        cd2 * v0, cd2 * v1, cd2 * v2,
        cd3 * (v0 * v0 - s3), cd3 * (v1 * v1 - s3),
        cd3 * (v0 * v1), cd3 * (v0 * v2), cd3 * (v1 * v2),
    ], 0)


def _edge_w(attr, ew2, v0c, v1c, v2c, dpwT, dpb):
    eb = 1600
    grid = (E // eb,)
    col = lambda: pl.BlockSpec((eb, 1), lambda i: (i, 0))
    return pl.pallas_call(
        _edge_w_kernel,
        grid=grid,
        in_specs=[
            pl.BlockSpec((eb, R), lambda i: (i, 0)),
            col(), col(), col(), col(),
            pl.BlockSpec((R, 3 * H), lambda i: (0, 0)),
            pl.BlockSpec((1, 3 * H), lambda i: (0, 0)),
        ],
        out_specs=pl.BlockSpec((NCOMP, eb, H), lambda i: (0, i, 0)),
        out_shape=jax.ShapeDtypeStruct((NCOMP, E, H), jnp.float32),
    )(attr, ew2, v0c, v1c, v2c, dpwT, dpb)


def _mul_block(buf, slot, wref):
    def body(i, _):
        for u in range(H // HC):
            sl = pl.ds(u * HC, HC)
            buf[slot, i, sl] = buf[slot, i, sl] * wref[slot, i, sl]
        return 0
    lax.fori_loop(0, KB, body, 0)


def _sc_scatter_body(src_hbm, dst_hbm, ptab_s, ptab_d, w_hbm, zeros_hbm,
                     out_hbm, q_hbm, idx_s, idx_d, qb, wb, acc_sh,
                     semq, semw, semsc, semg):
    cid = lax.axis_index("c")
    sid = lax.axis_index("s")
    wid = cid * NS + sid
    base_e = wid * ET
    n0 = sid * NPT

    pltpu.sync_copy(src_hbm.at[pl.ds(wid * NB_BLK, NB_BLK)], idx_s)

    for j in range(NCOMP):
        pltpu.sync_copy(zeros_hbm, acc_sh.at[pl.ds(n0, NPT)])
        plsc.subcore_barrier()

        if j == 0:
            def blk0(b, _):
                e0 = base_e + b * KB
                pltpu.sync_copy(dst_hbm.at[wid * NB_BLK + b], idx_d)
                c1 = pltpu.async_copy(ptab_s.at[idx_s.at[b, 0]],
                                      qb.at[0], semg)
                c2 = pltpu.async_copy(ptab_d.at[idx_d.at[0]],
                                      qb.at[1], semg)
                c1.wait()
                c2.wait()

                def qadd(i, _):
                    for u in range(H // HC):
                        sl = pl.ds(u * HC, HC)
                        qb[0, i, sl] = qb[0, i, sl] + qb[1, i, sl]
                    return 0
                lax.fori_loop(0, KB, qadd, 0)
                pltpu.sync_copy(qb.at[0], q_hbm.at[pl.ds(e0, KB)])
                pltpu.sync_copy(w_hbm.at[0, pl.ds(e0, KB)], wb.at[0])
                _mul_block(qb, 0, wb)
                pltpu.sync_copy(qb.at[0], acc_sh.at[idx_s.at[b, 0]], add=True)
                return 0
            lax.fori_loop(0, NB_BLK, blk0, 0)
        else:
            def issue_reads(b, slot):
                e0 = base_e + b * KB
                pltpu.async_copy(q_hbm.at[pl.ds(e0, KB)], qb.at[slot], semq)
                pltpu.async_copy(w_hbm.at[j, pl.ds(e0, KB)], wb.at[slot],
                                 semw)

            def wait_reads(b, slot):
                e0 = base_e + b * KB
                pltpu.make_async_copy(q_hbm.at[pl.ds(e0, KB)], qb.at[slot],
                                      semq).wait()
                pltpu.make_async_copy(w_hbm.at[j, pl.ds(e0, KB)],
                                      wb.at[slot], semw).wait()

            def issue_scatter(b, slot):
                pltpu.async_copy(qb.at[slot], acc_sh.at[idx_s.at[b, 0]],
                                 semsc, add=True)

            def wait_scatter(slot):
                pltpu.make_async_copy(qb.at[slot], acc_sh.at[idx_s.at[0, 0]],
                                      semsc).wait()

            issue_reads(0, 0)
            wait_reads(0, 0)
            issue_reads(1, 1)
            _mul_block(qb, 0, wb)
            issue_scatter(0, 0)

            def blk(b, _):
                cur = lax.rem(b, 2)
                wait_reads(b, cur)
                wait_scatter(1 - cur)
                issue_reads(b + 1, 1 - cur)
                _mul_block(qb, cur, wb)
                issue_scatter(b, cur)
                return 0
            lax.fori_loop(1, NB_BLK - 1, blk, 0)

            last = NB_BLK - 1
            lcur = last % 2
            wait_reads(last, lcur)
            _mul_block(qb, lcur, wb)
            wait_scatter(1 - lcur)
            issue_scatter(last, lcur)
            wait_scatter(lcur)
        plsc.subcore_barrier()

        pltpu.sync_copy(acc_sh.at[pl.ds(n0, NPT)],
                        out_hbm.at[cid, j, pl.ds(n0, NPT)])
        plsc.subcore_barrier()


def _sc_scatter(src3d, dst3d, ptab_s, ptab_d, w_hbm, zeros_npt):
    mesh = plsc.VectorSubcoreMesh(core_axis_name="c", subcore_axis_name="s",
                                  num_cores=NC)
    kern = functools.partial(
        pl.kernel,
        out_type=[jax.ShapeDtypeStruct((NC, NCOMP, NPAD, H), jnp.float32),
                  jax.ShapeDtypeStruct((E, H), jnp.float32)],
        mesh=mesh,
        scratch_types=[
            pltpu.VMEM((NB_BLK, 1, KS), jnp.int32),
            pltpu.VMEM((1, KS), jnp.int32),
            pltpu.VMEM((2, KB, H), jnp.float32),
            pltpu.VMEM((2, KB, H), jnp.float32),
            pltpu.VMEM_SHARED((NPAD, H), jnp.float32),
            pltpu.SemaphoreType.DMA,
            pltpu.SemaphoreType.DMA,
            pltpu.SemaphoreType.DMA,
            pltpu.SemaphoreType.DMA,
        ],
    )(_sc_scatter_body)
    return kern(src3d, dst3d, ptab_s, ptab_d, w_hbm, zeros_npt)


def _final_kernel(acc_ref, lt0T_ref, lt1T_ref, lt2T_ref, ls0T_ref, ls0b_ref,
                  ls1pT_ref, ls1bp_ref, g_ref, b_ref, out_ref):
    A = acc_ref[...]
    a = A[0] + A[1]
    si = a[0]
    va0, va1, va2 = a[1], a[2], a[3]
    m0, m1 = a[4], a[5]
    m2 = -(m0 + m1)
    m3, m4, m5 = a[6], a[7], a[8]
    tn = (3.0 * si * si
          + 2.0 * (va0 * va0 + va1 * va1 + va2 * va2)
          + m0 * m0 + m1 * m1 + m2 * m2
          + 2.0 * (m3 * m3 + m4 * m4 + m5 * m5))
    mu = jnp.mean(tn, axis=-1, keepdims=True)
    var = jnp.mean((tn - mu) ** 2, axis=-1, keepdims=True)
    ln = (tn - mu) * lax.rsqrt(var + 1e-5) * g_ref[...] + b_ref[...]

    def silu(x):
        return x / (1.0 + jnp.exp(-x))

    h1 = silu(_dot(ln, ls0T_ref[...]) + ls0b_ref[...])
    nrm = silu(_dot(h1, ls1pT_ref[...]) + ls1bp_ref[...])
    n0 = nrm[:, :H]
    n1 = nrm[:, H:2 * H]
    n2 = nrm[:, 2 * H:]

    lt0T = lt0T_ref[...]
    lt1T = lt1T_ref[...]
    lt2T = lt2T_ref[...]
    sip = _dot(si, lt0T)
    vap0 = _dot(va0, lt1T)
    vap1 = _dot(va1, lt1T)
    vap2 = _dot(va2, lt1T)
    mp0 = _dot(m0, lt2T)
    mp1 = _dot(m1, lt2T)
    mp2 = -(mp0 + mp1)
    mp3 = _dot(m3, lt2T)
    mp4 = _dot(m4, lt2T)
    mp5 = _dot(m5, lt2T)

    dsi = n0 * sip
    o00 = dsi + n2 * mp0
    o11 = dsi + n2 * mp1
    o22 = dsi + n2 * mp2
    o01 = n2 * mp3 - n1 * vap2
    o10 = n2 * mp3 + n1 * vap2
    o02 = n2 * mp4 + n1 * vap1
    o20 = n2 * mp4 - n1 * vap1
    o12 = n2 * mp5 - n1 * vap0
    o21 = n2 * mp5 + n1 * vap0
    out_ref[...] = jnp.stack(
        [o00, o01, o02, o10, o11, o12, o20, o21, o22], 0)


def _final(acc, lt0T, lt1T, lt2T, ls0T, ls0b, ls1pT, ls1bp, g2, b2):
    nb = 1024
    grid = (NPAD // nb,)
    full = lambda shape: pl.BlockSpec(shape, lambda i: (0,) * len(shape))
    return pl.pallas_call(
        _final_kernel,
        grid=grid,
        in_specs=[
            pl.BlockSpec((NC, NCOMP, nb, H), lambda i: (0, 0, i, 0)),
            full((H, H)), full((H, H)), full((H, H)),
            full((H, 2 * H)), full((1, 2 * H)),
            full((2 * H, 3 * H)), full((1, 3 * H)),
            full((1, H)), full((1, H)),
        ],
        out_specs=pl.BlockSpec((9, nb, H), lambda i: (0, i, 0)),
        out_shape=jax.ShapeDtypeStruct((9, NPAD, H), jnp.float32),
    )(acc, lt0T, lt1T, lt2T, ls0T, ls0b, ls1pT, ls1bp, g2, b2)


def kernel(z, edge_index, edge_weight, edge_vec_norm, edge_attr, labels,
           labels_mask, emb_w, emb2_w, emb2_b, label_emb_w, label_emb_b,
           label_emb2_w, label_emb2_b, mask_token, dp1_w, dp1_b, dp2_w,
           dp2_b, dp3_w, dp3_b, lt0_w, lt1_w, lt2_w, ls0_w, ls0_b, ls1_w,
           ls1_b, ln_g, ln_b):
    f32 = jnp.float32

    z2 = z.astype(jnp.int32).reshape(N, 1)
    lab2 = labels.astype(f32).reshape(N, 1)
    msk2 = labels_mask.astype(f32).reshape(N, 1)
    lwT = label_emb_w.astype(f32).reshape(1, H)
    leb = label_emb_b.astype(f32).reshape(1, H)
    mtok = mask_token.astype(f32).reshape(1, H)
    e2sT = emb2_w[:, :H].T.astype(f32)
    e2dT = emb2_w[:, H:].T.astype(f32)
    l2sT = label_emb2_w[:, :H].T.astype(f32)
    l2dT = label_emb2_w[:, H:].T.astype(f32)
    bq = (emb2_b + label_emb2_b).astype(f32).reshape(1, H)

    psrc, pdst = _node_tables(z2, lab2, msk2, emb_w.astype(f32), lwT, leb,
                              mtok, e2sT, e2dT, l2sT, l2dT, bq)

    dpwT = jnp.concatenate([dp1_w, dp2_w, dp3_w], 0).T.astype(f32)
    dpb = jnp.concatenate([dp1_b, dp2_b, dp3_b], 0).astype(f32).reshape(1, 3 * H)
    ew2 = edge_weight.astype(f32).reshape(E, 1)
    v0c = edge_vec_norm[:, 0].astype(f32).reshape(E, 1)
    v1c = edge_vec_norm[:, 1].astype(f32).reshape(E, 1)
    v2c = edge_vec_norm[:, 2].astype(f32).reshape(E, 1)
    w_hbm = _edge_w(edge_attr.astype(f32), ew2, v0c, v1c, v2c, dpwT, dpb)

    src3d = edge_index[0].astype(jnp.int32).reshape(E // KS, 1, KS)
    dst3d = edge_index[1].astype(jnp.int32).reshape(E // KS, 1, KS)
    zeros_npt = jnp.zeros((NPT, H), f32)
    acc, _q = _sc_scatter(src3d, dst3d, psrc, pdst, w_hbm, zeros_npt)

    lt0T = lt0_w.T.astype(f32)
    lt1T = lt1_w.T.astype(f32)
    lt2T = lt2_w.T.astype(f32)
    ls0T = ls0_w.T.astype(f32)
    ls0b = ls0_b.astype(f32).reshape(1, 2 * H)
    perm = np.array([h * 3 + k for k in range(3) for h in range(H)])
    ls1pT = ls1_w[perm].T.astype(f32)
    ls1bp = ls1_b[perm].astype(f32).reshape(1, 3 * H)
    g2 = ln_g.astype(f32).reshape(1, H)
    b2 = ln_b.astype(f32).reshape(1, H)

    out9 = _final(acc, lt0T, lt1T, lt2T, ls0T, ls0b, ls1pT, ls1bp, g2, b2)
    return out9[:, :N].transpose(1, 2, 0).reshape(N, H, 3, 3)
